# baseline (speedup 1.0000x reference)
import functools

import jax
import jax.numpy as jnp
from jax import lax
from jax.experimental import pallas as pl
from jax.experimental.pallas import tpu as pltpu
from jax.experimental.pallas import tpu_sc as plsc

N = 10000
D = 128
OUT = 16
HEADS = 8
E = 320000
Q = 50000

NC = 2
NS = 16
NW = NC * NS

ROWS_PER_TILE = N // NS
ECH = 80
EPT = E // NW
ENCH = EPT // ECH

QPAD = 50176
QPT = QPAD // NW
QCH = 112
QNCH = QPT // QCH

_MESH = plsc.VectorSubcoreMesh(core_axis_name="c", subcore_axis_name="s")
_SC_PARAMS = pltpu.CompilerParams(use_tc_tiling_on_sc=False,
                                  needs_layout_passes=False)


@functools.partial(
    pl.kernel,
    out_type=jax.ShapeDtypeStruct((NC * N, 16), jnp.float32),
    mesh=_MESH,
    compiler_params=_SC_PARAMS,
    scratch_types=[
        pltpu.VMEM((ENCH, ECH), jnp.int32),
        pltpu.VMEM((ECH, 16), jnp.float32),
        pltpu.VMEM_SHARED((N, 16), jnp.float32),
        pltpu.SemaphoreType.DMA,
        pltpu.SemaphoreType.DMA,
        pltpu.SemaphoreType.DMA,
        pltpu.SemaphoreType.DMA,
    ],
)
def _deg_kernel(dst2_hbm, zeros_hbm, ones_hbm, out_hbm, dst2_v, ones_v,
                acc_sh, sem0, sem1, sem2, sem3):
    c = lax.axis_index("c")
    s = lax.axis_index("s")
    r0 = s * ROWS_PER_TILE
    b80 = (c * (E // NC) + s * EPT) // ECH
    for k in range(5):
        pltpu.async_copy(zeros_hbm, acc_sh.at[pl.ds(r0 + 125 * k, 125)],
                         sem0)
    pltpu.async_copy(ones_hbm, ones_v, sem1)
    pltpu.async_copy(dst2_hbm.at[pl.ds(b80, ENCH)], dst2_v, sem2)
    for k in range(5):
        pltpu.make_async_copy(zeros_hbm,
                              acc_sh.at[pl.ds(r0 + 125 * k, 125)],
                              sem0).wait()
    pltpu.make_async_copy(ones_hbm, ones_v, sem1).wait()
    pltpu.make_async_copy(dst2_hbm.at[pl.ds(b80, ENCH)], dst2_v,
                          sem2).wait()
    plsc.subcore_barrier()

    ssem = (sem0, sem1, sem2, sem3)

    def body(g, _):
        c0 = 4 * g
        for k in range(4):
            @pl.when(g > 0)
            def _():
                pltpu.make_async_copy(
                    ones_v, acc_sh.at[dst2_v.at[c0 + k - 4]], ssem[k]).wait()
            pltpu.async_copy(ones_v, acc_sh.at[dst2_v.at[c0 + k]], ssem[k],
                             add=True)
        return 0

    lax.fori_loop(0, 31, body, 0)
    pltpu.make_async_copy(ones_v, acc_sh.at[dst2_v.at[120]], ssem[0]).wait()
    pltpu.async_copy(ones_v, acc_sh.at[dst2_v.at[124]], ssem[0], add=True)
    for k in range(4):
        pltpu.make_async_copy(
            ones_v, acc_sh.at[dst2_v.at[120 + k]], ssem[k]).wait()
    plsc.subcore_barrier()
    pltpu.sync_copy(acc_sh.at[pl.ds(r0, ROWS_PER_TILE)],
                    out_hbm.at[pl.ds(c * N + r0, ROWS_PER_TILE)])


BCH = 128
NB = E // BCH
NBC = NB // NC
MAIN = NBC // NS


@functools.partial(
    pl.kernel,
    out_type=jax.ShapeDtypeStruct((NC * N, D), jnp.float32),
    mesh=_MESH,
    compiler_params=_SC_PARAMS,
    scratch_types=[
        pltpu.VMEM((MAIN, BCH), jnp.int32),
        pltpu.VMEM((BCH,), jnp.int32),
        pltpu.VMEM((BCH,), jnp.int32),
        pltpu.VMEM((BCH,), jnp.int32),
        pltpu.VMEM((BCH, D), jnp.float32),
        pltpu.VMEM((BCH, D), jnp.float32),
        pltpu.VMEM_SHARED((N, D), jnp.float32),
        pltpu.SemaphoreType.DMA,
        pltpu.SemaphoreType.DMA,
        pltpu.SemaphoreType.DMA,
        pltpu.SemaphoreType.DMA,
        pltpu.SemaphoreType.DMA,
        pltpu.SemaphoreType.DMA,
    ],
)
def _scatter_kernel(hs_hbm, src_hbm, d128_hbm, zeros_hbm, out_hbm,
                    dstp_v, dstx_v, sv0, sv1, rows0, rows1, acc_sh,
                    gsem0, gsem1, ssem0, ssem1, isem0, isem1):
    c = lax.axis_index("c")
    s = lax.axis_index("s")
    sv = (sv0, sv1)
    rows = (rows0, rows1)
    gsem = (gsem0, gsem1)
    ssem = (ssem0, ssem1)
    isem = (isem0, isem1)
    r0 = s * ROWS_PER_TILE
    cb = c * NBC + s * MAIN + jnp.minimum(s, 2)

    def srcdma(lc, k):
        pltpu.async_copy(src_hbm.at[pl.ds((cb + lc) * BCH, BCH)], sv[k],
                         isem[k])

    def iwait(lc, k):
        pltpu.make_async_copy(src_hbm.at[pl.ds((cb + lc) * BCH, BCH)],
                              sv[k], isem[k]).wait()

    def gather(lc, k):
        return pltpu.async_copy(hs_hbm.at[sv[k]], rows[k], gsem[k])

    def gwait(lc, k):
        pltpu.make_async_copy(hs_hbm.at[sv[k]], rows[k], gsem[k]).wait()

    def scat(lc, k):
        return pltpu.async_copy(rows[k], acc_sh.at[dstp_v.at[lc]], ssem[k],
                                add=True)

    def swait(lc, k):
        pltpu.make_async_copy(rows[k], acc_sh.at[dstp_v.at[lc]],
                              ssem[k]).wait()

    for k in range(5):
        pltpu.async_copy(zeros_hbm, acc_sh.at[pl.ds(r0 + 125 * k, 125)],
                         gsem0)
    pltpu.async_copy(d128_hbm.at[pl.ds(cb, MAIN)], dstp_v, gsem1)
    srcdma(0, 0)
    srcdma(1, 1)
    for k in range(5):
        pltpu.make_async_copy(zeros_hbm,
                              acc_sh.at[pl.ds(r0 + 125 * k, 125)],
                              gsem0).wait()
    pltpu.make_async_copy(d128_hbm.at[pl.ds(cb, MAIN)], dstp_v,
                          gsem1).wait()
    plsc.subcore_barrier()
    iwait(0, 0)
    gather(0, 0)
    iwait(1, 1)
    gather(1, 1)

    def body(g, _):
        for k in range(2):
            ch = 2 * g + k
            gwait(ch, k)

            @pl.when(g < MAIN // 2 - 1)
            def _():
                srcdma(ch + 2, k)
            scat(ch, k)

        @pl.when(g < MAIN // 2 - 1)
        def _():
            for k in range(2):
                ch = 2 * g + k
                swait(ch, k)
                iwait(ch + 2, k)
                gather(ch + 2, k)
        return 0

    lax.fori_loop(0, MAIN // 2, body, 0)
    swait(MAIN - 2, 0)
    swait(MAIN - 1, 1)

    @pl.when(s < 2)
    def _():
        pltpu.sync_copy(d128_hbm.at[cb + MAIN], dstx_v)
        pltpu.sync_copy(src_hbm.at[pl.ds((cb + MAIN) * BCH, BCH)], sv0)
        pltpu.async_copy(hs_hbm.at[sv0], rows0, gsem0).wait()
        pltpu.async_copy(rows0, acc_sh.at[dstx_v], ssem0, add=True).wait()

    plsc.subcore_barrier()
    pltpu.sync_copy(acc_sh.at[pl.ds(r0, ROWS_PER_TILE)],
                    out_hbm.at[pl.ds(c * N + r0, ROWS_PER_TILE)])


@functools.partial(
    pl.kernel,
    out_type=[
        jax.ShapeDtypeStruct((QPAD,), jnp.float32),
        jax.ShapeDtypeStruct((QPAD,), jnp.float32),
        jax.ShapeDtypeStruct((QPAD,), jnp.float32),
    ],
    mesh=_MESH,
    compiler_params=_SC_PARAMS,
    scratch_types=[
        pltpu.VMEM((N,), jnp.int32),
        pltpu.VMEM((2, QCH), jnp.int32),
        pltpu.VMEM((2, QCH), jnp.int32),
        pltpu.VMEM((2, 4, QCH), jnp.int32),
        pltpu.VMEM((2, 4, QCH, OUT), jnp.float32),
        pltpu.VMEM((QPT,), jnp.float32),
        pltpu.VMEM((QPT,), jnp.float32),
        pltpu.VMEM((QPT,), jnp.float32),
        pltpu.SemaphoreType.DMA,
        pltpu.SemaphoreType.DMA,
        pltpu.SemaphoreType.DMA,
        pltpu.SemaphoreType.DMA,
    ],
)
def _query_kernel(ef_hbm, y_hbm, qr_hbm, qc_hbm,
                  pi_hbm, pj_hbm, pr_hbm,
                  y_v, qr_v, qc_v, idx_v, rows_v, pi_v, pj_v, pr_v,
                  qsem0, qsem1, qdsem0, qdsem1):
    c = lax.axis_index("c")
    s = lax.axis_index("s")
    qsem = (qsem0, qsem1)
    qdsem = (qdsem0, qdsem1)
    wid = s * NC + c
    qbase = wid * QPT
    pltpu.sync_copy(y_hbm, y_v)

    def qdma(ch, k):
        qoff = qbase + ch * QCH
        pltpu.async_copy(qr_hbm.at[pl.ds(qoff, QCH)], qr_v.at[k], qdsem[k])
        pltpu.async_copy(qc_hbm.at[pl.ds(qoff, QCH)], qc_v.at[k], qdsem[k])

    def qwait(ch, k):
        qoff = qbase + ch * QCH
        pltpu.make_async_copy(qr_hbm.at[pl.ds(qoff, QCH)], qr_v.at[k],
                              qdsem[k]).wait()
        pltpu.make_async_copy(qc_hbm.at[pl.ds(qoff, QCH)], qc_v.at[k],
                              qdsem[k]).wait()

    def build_fire(k):
        for j in range(QCH // 16):
            qr16 = qr_v[k, pl.ds(16 * j, 16)]
            qc16 = qc_v[k, pl.ds(16 * j, 16)]
            yr = plsc.load_gather(y_v, [qr16])
            yc = plsc.load_gather(y_v, [qc16])
            idx_v[k, 0, pl.ds(16 * j, 16)] = qr16 * HEADS + yr
            idx_v[k, 1, pl.ds(16 * j, 16)] = qc16 * HEADS + yr
            idx_v[k, 2, pl.ds(16 * j, 16)] = qr16 * HEADS + yc
            idx_v[k, 3, pl.ds(16 * j, 16)] = qc16 * HEADS + yc
        for i in range(4):
            pltpu.async_copy(ef_hbm.at[idx_v.at[k, i]], rows_v.at[k, i],
                             qsem[k])

    def compute(ch, k):
        for i in range(4):
            pltpu.make_async_copy(ef_hbm.at[idx_v.at[k, i]],
                                  rows_v.at[k, i], qsem[k]).wait()
        k16 = jnp.full((16,), k, jnp.int32)
        i16 = [jnp.full((16,), i, jnp.int32) for i in range(4)]

        def jbody(j, _):
            row16 = lax.iota(jnp.int32, 16) + 16 * j
            acc_i = jnp.zeros((16,), jnp.float32)
            acc_j = jnp.zeros((16,), jnp.float32)
            for f in range(OUT):
                f16 = jnp.full((16,), f, jnp.int32)
                va = plsc.load_gather(rows_v, [k16, i16[0], row16, f16])
                vb = plsc.load_gather(rows_v, [k16, i16[1], row16, f16])
                vc = plsc.load_gather(rows_v, [k16, i16[2], row16, f16])
                vd = plsc.load_gather(rows_v, [k16, i16[3], row16, f16])
                acc_i = acc_i + va * vb
                acc_j = acc_j + vc * vd
            o = ch * QCH + 16 * j
            pi_v[pl.ds(o, 16)] = acc_i
            pj_v[pl.ds(o, 16)] = acc_j
            pr_v[pl.ds(o, 16)] = (acc_i + acc_j) * 0.5
            return 0

        lax.fori_loop(0, QCH // 16, jbody, 0)

    qdma(0, 0)
    qwait(0, 0)
    build_fire(0)
    qdma(1, 1)
    qwait(1, 1)
    build_fire(1)

    def body(gp, _):
        ch0 = 2 * gp

        @pl.when(gp < QNCH // 2 - 1)
        def _():
            qdma(ch0 + 2, 0)
        compute(ch0, 0)

        @pl.when(gp < QNCH // 2 - 1)
        def _():
            qwait(ch0 + 2, 0)
            build_fire(0)
            qdma(ch0 + 3, 1)
        compute(ch0 + 1, 1)

        @pl.when(gp < QNCH // 2 - 1)
        def _():
            qwait(ch0 + 3, 1)
            build_fire(1)
        return 0

    lax.fori_loop(0, QNCH // 2, body, 0)
    pltpu.sync_copy(pi_v, pi_hbm.at[pl.ds(qbase, QPT)])
    pltpu.sync_copy(pj_v, pj_hbm.at[pl.ds(qbase, QPT)])
    pltpu.sync_copy(pr_v, pr_hbm.at[pl.ds(qbase, QPT)])


_BLK = 2000
_GRID = N // _BLK


def _mm_scale_body(dp0_ref, dp1_ref, x_ref, w_ref, hs_ref, dinv_ref):
    deg = jnp.sum(dp0_ref[...] + dp1_ref[...], axis=1) * (1.0 / 16.0) + 1.0
    dinv = lax.rsqrt(deg)
    h = jnp.dot(x_ref[...], w_ref[...], preferred_element_type=jnp.float32)
    hs_ref[...] = h * dinv[:, None]
    dinv_ref[...] = dinv[:, None]


def _mid_body(a0_ref, a1_ref, hs_ref, dinv_ref, b_ref, w_ref,
              hid_ref, hs2_ref):
    dinv = dinv_ref[...]
    pre = (a0_ref[...] + a1_ref[...] + hs_ref[...]) * dinv + b_ref[...]
    hid = jnp.maximum(pre, 0.0)
    hid_ref[...] = hid
    h2 = jnp.dot(hid, w_ref[...], preferred_element_type=jnp.float32)
    hs2_ref[...] = h2 * dinv


def _emb_body(a0_ref, a1_ref, hs_ref, dinv_ref, b_ref, emb_ref):
    emb_ref[...] = (a0_ref[...] + a1_ref[...] + hs_ref[...]) * dinv_ref[...] \
        + b_ref[...]


def _tc_mm_scale(degp, x, w):
    return pl.pallas_call(
        _mm_scale_body,
        grid=(_GRID,),
        in_specs=[
            pl.BlockSpec((_BLK, 16), lambda g: (g, 0)),
            pl.BlockSpec((_BLK, 16), lambda g: (g + _GRID, 0)),
            pl.BlockSpec((_BLK, D), lambda g: (g, 0)),
            pl.BlockSpec((D, D), lambda g: (0, 0)),
        ],
        out_specs=[
            pl.BlockSpec((_BLK, D), lambda g: (g, 0)),
            pl.BlockSpec((_BLK, 1), lambda g: (g, 0)),
        ],
        out_shape=[
            jax.ShapeDtypeStruct((N, D), jnp.float32),
            jax.ShapeDtypeStruct((N, 1), jnp.float32),
        ],
    )(degp, degp, x, w)


def _tc_mid(accp, hs1, dinv, b1, w2):
    return pl.pallas_call(
        _mid_body,
        grid=(_GRID,),
        in_specs=[
            pl.BlockSpec((_BLK, D), lambda g: (g, 0)),
            pl.BlockSpec((_BLK, D), lambda g: (g + _GRID, 0)),
            pl.BlockSpec((_BLK, D), lambda g: (g, 0)),
            pl.BlockSpec((_BLK, 1), lambda g: (g, 0)),
            pl.BlockSpec((1, D), lambda g: (0, 0)),
            pl.BlockSpec((D, D), lambda g: (0, 0)),
        ],
        out_specs=[
            pl.BlockSpec((_BLK, D), lambda g: (g, 0)),
            pl.BlockSpec((_BLK, D), lambda g: (g, 0)),
        ],
        out_shape=[
            jax.ShapeDtypeStruct((N, D), jnp.float32),
            jax.ShapeDtypeStruct((N, D), jnp.float32),
        ],
    )(accp, accp, hs1, dinv, b1, w2)


def _tc_emb(accp, hs2, dinv, b2):
    return pl.pallas_call(
        _emb_body,
        grid=(_GRID,),
        in_specs=[
            pl.BlockSpec((_BLK, D), lambda g: (g, 0)),
            pl.BlockSpec((_BLK, D), lambda g: (g + _GRID, 0)),
            pl.BlockSpec((_BLK, D), lambda g: (g, 0)),
            pl.BlockSpec((_BLK, 1), lambda g: (g, 0)),
            pl.BlockSpec((1, D), lambda g: (0, 0)),
        ],
        out_specs=pl.BlockSpec((_BLK, D), lambda g: (g, 0)),
        out_shape=jax.ShapeDtypeStruct((N, D), jnp.float32),
    )(accp, accp, hs2, dinv, b2)


@jax.jit
def _run(x, edge_index, y, q_edge_index, W1, b1, W2, b2):
    src2 = edge_index[0].reshape(E // ECH, ECH)
    dst2 = edge_index[1].reshape(E // ECH, ECH)
    dst128 = edge_index[1].reshape(E // BCH, BCH)
    zeros = jnp.zeros((125, D), jnp.float32)
    zeros16 = jnp.zeros((125, 16), jnp.float32)
    ones16 = jnp.ones((ECH, 16), jnp.float32)

    degp = _deg_kernel(dst2, zeros16, ones16)
    hs1, dinv = _tc_mm_scale(degp, x, W1)
    acc1 = _scatter_kernel(hs1, edge_index[0], dst128, zeros)
    hiddens, hs2 = _tc_mid(acc1, hs1, dinv, b1.reshape(1, D), W2)
    acc2 = _scatter_kernel(hs2, edge_index[0], dst128, zeros)
    emb2d = _tc_emb(acc2, hs2, dinv, b2.reshape(1, D))

    ef = emb2d.reshape(N * HEADS, OUT)
    qpad = jnp.zeros((2, QPAD), jnp.int32).at[:, :Q].set(q_edge_index)
    pi, pj, pr = _query_kernel(ef, y, qpad[0], qpad[1])
    emb = emb2d.reshape(N, HEADS, OUT)
    return hiddens, emb, pi[:Q], pj[:Q], pr[:Q]


def kernel(x, edge_index, y, q_edge_index, W1, b1, W2, b2):
    return _run(x, edge_index, y, q_edge_index, W1, b1, W2, b2)


# speedup vs baseline: 1.1107x; 1.1107x over previous
import functools

import jax
import jax.numpy as jnp
from jax import lax
from jax.experimental import pallas as pl
from jax.experimental.pallas import tpu as pltpu
from jax.experimental.pallas import tpu_sc as plsc

N = 10000
D = 128
OUT = 16
HEADS = 8
E = 320000
Q = 50000

NC = 2
NS = 16
NW = NC * NS

ROWS_PER_TILE = N // NS
ECH = 80
EPT = E // NW
ENCH = EPT // ECH

QPAD = 50176
QPT = QPAD // NW
QCH = 112
QNCH = QPT // QCH

_MESH = plsc.VectorSubcoreMesh(core_axis_name="c", subcore_axis_name="s")
_SC_PARAMS = pltpu.CompilerParams(use_tc_tiling_on_sc=False,
                                  needs_layout_passes=False)


@functools.partial(
    pl.kernel,
    out_type=jax.ShapeDtypeStruct((NC * N, 16), jnp.float32),
    mesh=_MESH,
    compiler_params=_SC_PARAMS,
    scratch_types=[
        pltpu.VMEM((ENCH, ECH), jnp.int32),
        pltpu.VMEM((ECH, 16), jnp.float32),
        pltpu.VMEM_SHARED((N, 16), jnp.float32),
        pltpu.SemaphoreType.DMA,
        pltpu.SemaphoreType.DMA,
        pltpu.SemaphoreType.DMA,
        pltpu.SemaphoreType.DMA,
    ],
)
def _deg_kernel(dst2_hbm, zeros_hbm, ones_hbm, out_hbm, dst2_v, ones_v,
                acc_sh, sem0, sem1, sem2, sem3):
    c = lax.axis_index("c")
    s = lax.axis_index("s")
    r0 = s * ROWS_PER_TILE
    b80 = (c * (E // NC) + s * EPT) // ECH
    for k in range(5):
        pltpu.async_copy(zeros_hbm, acc_sh.at[pl.ds(r0 + 125 * k, 125)],
                         sem0)
    pltpu.async_copy(ones_hbm, ones_v, sem1)
    pltpu.async_copy(dst2_hbm.at[pl.ds(b80, ENCH)], dst2_v, sem2)
    for k in range(5):
        pltpu.make_async_copy(zeros_hbm,
                              acc_sh.at[pl.ds(r0 + 125 * k, 125)],
                              sem0).wait()
    pltpu.make_async_copy(ones_hbm, ones_v, sem1).wait()
    pltpu.make_async_copy(dst2_hbm.at[pl.ds(b80, ENCH)], dst2_v,
                          sem2).wait()
    plsc.subcore_barrier()

    ssem = (sem0, sem1, sem2, sem3)

    def body(g, _):
        c0 = 4 * g
        for k in range(4):
            @pl.when(g > 0)
            def _():
                pltpu.make_async_copy(
                    ones_v, acc_sh.at[dst2_v.at[c0 + k - 4]], ssem[k]).wait()
            pltpu.async_copy(ones_v, acc_sh.at[dst2_v.at[c0 + k]], ssem[k],
                             add=True)
        return 0

    lax.fori_loop(0, 31, body, 0)
    pltpu.make_async_copy(ones_v, acc_sh.at[dst2_v.at[120]], ssem[0]).wait()
    pltpu.async_copy(ones_v, acc_sh.at[dst2_v.at[124]], ssem[0], add=True)
    for k in range(4):
        pltpu.make_async_copy(
            ones_v, acc_sh.at[dst2_v.at[120 + k]], ssem[k]).wait()
    plsc.subcore_barrier()
    pltpu.sync_copy(acc_sh.at[pl.ds(r0, ROWS_PER_TILE)],
                    out_hbm.at[pl.ds(c * N + r0, ROWS_PER_TILE)])


@functools.partial(
    pl.kernel,
    out_type=jax.ShapeDtypeStruct((NC * N, D), jnp.float32),
    mesh=_MESH,
    compiler_params=_SC_PARAMS,
    scratch_types=[
        pltpu.VMEM((ENCH, ECH), jnp.int32),
        pltpu.VMEM((ENCH, ECH), jnp.int32),
        pltpu.VMEM((ECH, D), jnp.float32),
        pltpu.VMEM((ECH, D), jnp.float32),
        pltpu.VMEM((ECH, D), jnp.float32),
        pltpu.VMEM_SHARED((N, D), jnp.float32),
        pltpu.SemaphoreType.DMA,
        pltpu.SemaphoreType.DMA,
        pltpu.SemaphoreType.DMA,
        pltpu.SemaphoreType.DMA,
        pltpu.SemaphoreType.DMA,
        pltpu.SemaphoreType.DMA,
    ],
)
def _scatter_kernel(hs_hbm, src2_hbm, dst2_hbm, zeros_hbm, out_hbm,
                    src2_v, dst2_v, rows0, rows1, rows2, acc_sh,
                    gsem0, gsem1, gsem2, ssem0, ssem1, ssem2):
    c = lax.axis_index("c")
    s = lax.axis_index("s")
    rows = (rows0, rows1, rows2)
    gsem = (gsem0, gsem1, gsem2)
    ssem = (ssem0, ssem1, ssem2)
    r0 = s * ROWS_PER_TILE
    b80 = (c * (E // NC) + s * EPT) // ECH
    for k in range(5):
        pltpu.async_copy(zeros_hbm, acc_sh.at[pl.ds(r0 + 125 * k, 125)],
                         gsem0)
    pltpu.async_copy(src2_hbm.at[pl.ds(b80, ENCH)], src2_v, gsem1)
    pltpu.async_copy(dst2_hbm.at[pl.ds(b80, ENCH)], dst2_v, gsem2)
    for k in range(5):
        pltpu.make_async_copy(zeros_hbm,
                              acc_sh.at[pl.ds(r0 + 125 * k, 125)],
                              gsem0).wait()
    pltpu.make_async_copy(src2_hbm.at[pl.ds(b80, ENCH)], src2_v,
                          gsem1).wait()
    pltpu.make_async_copy(dst2_hbm.at[pl.ds(b80, ENCH)], dst2_v,
                          gsem2).wait()
    plsc.subcore_barrier()

    def gather(ch, k):
        return pltpu.async_copy(hs_hbm.at[src2_v.at[ch]], rows[k], gsem[k])

    def gwait(ch, k):
        pltpu.make_async_copy(hs_hbm.at[src2_v.at[ch]], rows[k],
                              gsem[k]).wait()

    def scat(ch, k):
        return pltpu.async_copy(rows[k], acc_sh.at[dst2_v.at[ch]], ssem[k],
                                add=True)

    def swait(ch, k):
        pltpu.make_async_copy(rows[k], acc_sh.at[dst2_v.at[ch]],
                              ssem[k]).wait()

    for ch in range(2):
        gather(ch, 0).wait()
        scat(ch, 0)
        swait(ch, 0)
    for k in range(3):
        gather(2 + k, k)

    def body(g, _):
        c0 = 3 * g + 2
        for k in range(3):
            gwait(c0 + k, k)
            scat(c0 + k, k)

        @pl.when(g < 40)
        def _():
            for k in range(3):
                swait(c0 + k, k)
                gather(c0 + 3 + k, k)
        return 0

    lax.fori_loop(0, 41, body, 0)
    for k in range(3):
        swait(122 + k, k)
    plsc.subcore_barrier()
    pltpu.sync_copy(acc_sh.at[pl.ds(r0, ROWS_PER_TILE)],
                    out_hbm.at[pl.ds(c * N + r0, ROWS_PER_TILE)])


@functools.partial(
    pl.kernel,
    out_type=[
        jax.ShapeDtypeStruct((QPAD,), jnp.float32),
        jax.ShapeDtypeStruct((QPAD,), jnp.float32),
        jax.ShapeDtypeStruct((QPAD,), jnp.float32),
    ],
    mesh=_MESH,
    compiler_params=_SC_PARAMS,
    scratch_types=[
        pltpu.VMEM((N,), jnp.int32),
        pltpu.VMEM((2, QCH), jnp.int32),
        pltpu.VMEM((2, QCH), jnp.int32),
        pltpu.VMEM((2, 4, QCH), jnp.int32),
        pltpu.VMEM((2, 4, QCH, OUT), jnp.float32),
        pltpu.VMEM((QPT,), jnp.float32),
        pltpu.VMEM((QPT,), jnp.float32),
        pltpu.VMEM((QPT,), jnp.float32),
        pltpu.SemaphoreType.DMA,
        pltpu.SemaphoreType.DMA,
        pltpu.SemaphoreType.DMA,
        pltpu.SemaphoreType.DMA,
    ],
)
def _query_kernel(ef_hbm, y_hbm, qr_hbm, qc_hbm,
                  pi_hbm, pj_hbm, pr_hbm,
                  y_v, qr_v, qc_v, idx_v, rows_v, pi_v, pj_v, pr_v,
                  qsem0, qsem1, qdsem0, qdsem1):
    c = lax.axis_index("c")
    s = lax.axis_index("s")
    qsem = (qsem0, qsem1)
    qdsem = (qdsem0, qdsem1)
    wid = s * NC + c
    qbase = wid * QPT
    pltpu.sync_copy(y_hbm, y_v)

    def qdma(ch, k):
        qoff = qbase + ch * QCH
        pltpu.async_copy(qr_hbm.at[pl.ds(qoff, QCH)], qr_v.at[k], qdsem[k])
        pltpu.async_copy(qc_hbm.at[pl.ds(qoff, QCH)], qc_v.at[k], qdsem[k])

    def qwait(ch, k):
        qoff = qbase + ch * QCH
        pltpu.make_async_copy(qr_hbm.at[pl.ds(qoff, QCH)], qr_v.at[k],
                              qdsem[k]).wait()
        pltpu.make_async_copy(qc_hbm.at[pl.ds(qoff, QCH)], qc_v.at[k],
                              qdsem[k]).wait()

    def build_fire(k):
        for j in range(QCH // 16):
            qr16 = qr_v[k, pl.ds(16 * j, 16)]
            qc16 = qc_v[k, pl.ds(16 * j, 16)]
            yr = plsc.load_gather(y_v, [qr16])
            yc = plsc.load_gather(y_v, [qc16])
            idx_v[k, 0, pl.ds(16 * j, 16)] = qr16 * HEADS + yr
            idx_v[k, 1, pl.ds(16 * j, 16)] = qc16 * HEADS + yr
            idx_v[k, 2, pl.ds(16 * j, 16)] = qr16 * HEADS + yc
            idx_v[k, 3, pl.ds(16 * j, 16)] = qc16 * HEADS + yc
        for i in range(4):
            pltpu.async_copy(ef_hbm.at[idx_v.at[k, i]], rows_v.at[k, i],
                             qsem[k])

    def compute(ch, k):
        for i in range(4):
            pltpu.make_async_copy(ef_hbm.at[idx_v.at[k, i]],
                                  rows_v.at[k, i], qsem[k]).wait()
        k16 = jnp.full((16,), k, jnp.int32)
        i16 = [jnp.full((16,), i, jnp.int32) for i in range(4)]

        def jbody(j, _):
            row16 = lax.iota(jnp.int32, 16) + 16 * j
            acc_i = jnp.zeros((16,), jnp.float32)
            acc_j = jnp.zeros((16,), jnp.float32)
            for f in range(OUT):
                f16 = jnp.full((16,), f, jnp.int32)
                va = plsc.load_gather(rows_v, [k16, i16[0], row16, f16])
                vb = plsc.load_gather(rows_v, [k16, i16[1], row16, f16])
                vc = plsc.load_gather(rows_v, [k16, i16[2], row16, f16])
                vd = plsc.load_gather(rows_v, [k16, i16[3], row16, f16])
                acc_i = acc_i + va * vb
                acc_j = acc_j + vc * vd
            o = ch * QCH + 16 * j
            pi_v[pl.ds(o, 16)] = acc_i
            pj_v[pl.ds(o, 16)] = acc_j
            pr_v[pl.ds(o, 16)] = (acc_i + acc_j) * 0.5
            return 0

        lax.fori_loop(0, QCH // 16, jbody, 0)

    qdma(0, 0)
    qwait(0, 0)
    build_fire(0)
    qdma(1, 1)
    qwait(1, 1)
    build_fire(1)

    def body(gp, _):
        ch0 = 2 * gp

        @pl.when(gp < QNCH // 2 - 1)
        def _():
            qdma(ch0 + 2, 0)
        compute(ch0, 0)

        @pl.when(gp < QNCH // 2 - 1)
        def _():
            qwait(ch0 + 2, 0)
            build_fire(0)
            qdma(ch0 + 3, 1)
        compute(ch0 + 1, 1)

        @pl.when(gp < QNCH // 2 - 1)
        def _():
            qwait(ch0 + 3, 1)
            build_fire(1)
        return 0

    lax.fori_loop(0, QNCH // 2, body, 0)
    pltpu.sync_copy(pi_v, pi_hbm.at[pl.ds(qbase, QPT)])
    pltpu.sync_copy(pj_v, pj_hbm.at[pl.ds(qbase, QPT)])
    pltpu.sync_copy(pr_v, pr_hbm.at[pl.ds(qbase, QPT)])


_BLK = 2000
_GRID = N // _BLK


def _mm_scale_body(dp0_ref, dp1_ref, x_ref, w_ref, hs_ref, dinv_ref):
    deg = jnp.sum(dp0_ref[...] + dp1_ref[...], axis=1) * (1.0 / 16.0) + 1.0
    dinv = lax.rsqrt(deg)
    h = jnp.dot(x_ref[...], w_ref[...], preferred_element_type=jnp.float32)
    hs_ref[...] = h * dinv[:, None]
    dinv_ref[...] = dinv[:, None]


def _mid_body(a0_ref, a1_ref, hs_ref, dinv_ref, b_ref, w_ref,
              hid_ref, hs2_ref):
    dinv = dinv_ref[...]
    pre = (a0_ref[...] + a1_ref[...] + hs_ref[...]) * dinv + b_ref[...]
    hid = jnp.maximum(pre, 0.0)
    hid_ref[...] = hid
    h2 = jnp.dot(hid, w_ref[...], preferred_element_type=jnp.float32)
    hs2_ref[...] = h2 * dinv


def _emb_body(a0_ref, a1_ref, hs_ref, dinv_ref, b_ref, emb_ref):
    emb_ref[...] = (a0_ref[...] + a1_ref[...] + hs_ref[...]) * dinv_ref[...] \
        + b_ref[...]


def _tc_mm_scale(degp, x, w):
    return pl.pallas_call(
        _mm_scale_body,
        grid=(_GRID,),
        in_specs=[
            pl.BlockSpec((_BLK, 16), lambda g: (g, 0)),
            pl.BlockSpec((_BLK, 16), lambda g: (g + _GRID, 0)),
            pl.BlockSpec((_BLK, D), lambda g: (g, 0)),
            pl.BlockSpec((D, D), lambda g: (0, 0)),
        ],
        out_specs=[
            pl.BlockSpec((_BLK, D), lambda g: (g, 0)),
            pl.BlockSpec((_BLK, 1), lambda g: (g, 0)),
        ],
        out_shape=[
            jax.ShapeDtypeStruct((N, D), jnp.float32),
            jax.ShapeDtypeStruct((N, 1), jnp.float32),
        ],
    )(degp, degp, x, w)


def _tc_mid(accp, hs1, dinv, b1, w2):
    return pl.pallas_call(
        _mid_body,
        grid=(_GRID,),
        in_specs=[
            pl.BlockSpec((_BLK, D), lambda g: (g, 0)),
            pl.BlockSpec((_BLK, D), lambda g: (g + _GRID, 0)),
            pl.BlockSpec((_BLK, D), lambda g: (g, 0)),
            pl.BlockSpec((_BLK, 1), lambda g: (g, 0)),
            pl.BlockSpec((1, D), lambda g: (0, 0)),
            pl.BlockSpec((D, D), lambda g: (0, 0)),
        ],
        out_specs=[
            pl.BlockSpec((_BLK, D), lambda g: (g, 0)),
            pl.BlockSpec((_BLK, D), lambda g: (g, 0)),
        ],
        out_shape=[
            jax.ShapeDtypeStruct((N, D), jnp.float32),
            jax.ShapeDtypeStruct((N, D), jnp.float32),
        ],
    )(accp, accp, hs1, dinv, b1, w2)


def _tc_emb(accp, hs2, dinv, b2):
    return pl.pallas_call(
        _emb_body,
        grid=(_GRID,),
        in_specs=[
            pl.BlockSpec((_BLK, D), lambda g: (g, 0)),
            pl.BlockSpec((_BLK, D), lambda g: (g + _GRID, 0)),
            pl.BlockSpec((_BLK, D), lambda g: (g, 0)),
            pl.BlockSpec((_BLK, 1), lambda g: (g, 0)),
            pl.BlockSpec((1, D), lambda g: (0, 0)),
        ],
        out_specs=pl.BlockSpec((_BLK, D), lambda g: (g, 0)),
        out_shape=jax.ShapeDtypeStruct((N, D), jnp.float32),
    )(accp, accp, hs2, dinv, b2)


@jax.jit
def _run(x, edge_index, y, q_edge_index, W1, b1, W2, b2):
    src2 = edge_index[0].reshape(E // ECH, ECH)
    dst2 = edge_index[1].reshape(E // ECH, ECH)
    zeros = jnp.zeros((125, D), jnp.float32)
    zeros16 = jnp.zeros((125, 16), jnp.float32)
    ones16 = jnp.ones((ECH, 16), jnp.float32)

    degp = _deg_kernel(dst2, zeros16, ones16)
    hs1, dinv = _tc_mm_scale(degp, x, W1)
    acc1 = _scatter_kernel(hs1, src2, dst2, zeros)
    hiddens, hs2 = _tc_mid(acc1, hs1, dinv, b1.reshape(1, D), W2)
    acc2 = _scatter_kernel(hs2, src2, dst2, zeros)
    emb2d = _tc_emb(acc2, hs2, dinv, b2.reshape(1, D))

    ef = emb2d.reshape(N * HEADS, OUT)
    qpad = jnp.zeros((2, QPAD), jnp.int32).at[:, :Q].set(q_edge_index)
    pi, pj, pr = _query_kernel(ef, y, qpad[0], qpad[1])
    emb = emb2d.reshape(N, HEADS, OUT)
    return hiddens, emb, pi[:Q], pj[:Q], pr[:Q]


def kernel(x, edge_index, y, q_edge_index, W1, b1, W2, b2):
    return _run(x, edge_index, y, q_edge_index, W1, b1, W2, b2)


# speedup vs baseline: 1.1488x; 1.0342x over previous
import functools

import jax
import jax.numpy as jnp
from jax import lax
from jax.experimental import pallas as pl
from jax.experimental.pallas import tpu as pltpu
from jax.experimental.pallas import tpu_sc as plsc

N = 10000
D = 128
OUT = 16
HEADS = 8
E = 320000
Q = 50000

NC = 2
NS = 16
NW = NC * NS

ROWS_PER_TILE = N // NS
ECH = 80
EPT = E // NW
ENCH = EPT // ECH

QPAD = 50176
QPT = QPAD // NW
QCH = 112
QNCH = QPT // QCH

_MESH = plsc.VectorSubcoreMesh(core_axis_name="c", subcore_axis_name="s")
_SC_PARAMS = pltpu.CompilerParams(use_tc_tiling_on_sc=False,
                                  needs_layout_passes=False)


@functools.partial(
    pl.kernel,
    out_type=jax.ShapeDtypeStruct((NC * N, 16), jnp.float32),
    mesh=_MESH,
    compiler_params=_SC_PARAMS,
    scratch_types=[
        pltpu.VMEM((ENCH, ECH), jnp.int32),
        pltpu.VMEM((ECH, 16), jnp.float32),
        pltpu.VMEM_SHARED((N, 16), jnp.float32),
        pltpu.SemaphoreType.DMA,
        pltpu.SemaphoreType.DMA,
        pltpu.SemaphoreType.DMA,
        pltpu.SemaphoreType.DMA,
    ],
)
def _deg_kernel(dst2_hbm, zeros_hbm, ones_hbm, out_hbm, dst2_v, ones_v,
                acc_sh, sem0, sem1, sem2, sem3):
    c = lax.axis_index("c")
    s = lax.axis_index("s")
    r0 = s * ROWS_PER_TILE
    b80 = (c * (E // NC) + s * EPT) // ECH
    for k in range(5):
        pltpu.async_copy(zeros_hbm, acc_sh.at[pl.ds(r0 + 125 * k, 125)],
                         sem0)
    pltpu.async_copy(ones_hbm, ones_v, sem1)
    pltpu.async_copy(dst2_hbm.at[pl.ds(b80, ENCH)], dst2_v, sem2)
    for k in range(5):
        pltpu.make_async_copy(zeros_hbm,
                              acc_sh.at[pl.ds(r0 + 125 * k, 125)],
                              sem0).wait()
    pltpu.make_async_copy(ones_hbm, ones_v, sem1).wait()
    pltpu.make_async_copy(dst2_hbm.at[pl.ds(b80, ENCH)], dst2_v,
                          sem2).wait()
    plsc.subcore_barrier()

    ssem = (sem0, sem1, sem2, sem3)

    def body(g, _):
        c0 = 4 * g
        for k in range(4):
            @pl.when(g > 0)
            def _():
                pltpu.make_async_copy(
                    ones_v, acc_sh.at[dst2_v.at[c0 + k - 4]], ssem[k]).wait()
            pltpu.async_copy(ones_v, acc_sh.at[dst2_v.at[c0 + k]], ssem[k],
                             add=True)
        return 0

    lax.fori_loop(0, 31, body, 0)
    pltpu.make_async_copy(ones_v, acc_sh.at[dst2_v.at[120]], ssem[0]).wait()
    pltpu.async_copy(ones_v, acc_sh.at[dst2_v.at[124]], ssem[0], add=True)
    for k in range(4):
        pltpu.make_async_copy(
            ones_v, acc_sh.at[dst2_v.at[120 + k]], ssem[k]).wait()
    plsc.subcore_barrier()
    pltpu.sync_copy(acc_sh.at[pl.ds(r0, ROWS_PER_TILE)],
                    out_hbm.at[pl.ds(c * N + r0, ROWS_PER_TILE)])


@functools.partial(
    pl.kernel,
    out_type=jax.ShapeDtypeStruct((NC * N, D), jnp.float32),
    mesh=_MESH,
    compiler_params=_SC_PARAMS,
    scratch_types=[
        pltpu.VMEM((45, ECH), jnp.int32),
        pltpu.VMEM((45, ECH), jnp.int32),
        pltpu.VMEM((ECH, D), jnp.float32),
        pltpu.VMEM((ECH, D), jnp.float32),
        pltpu.VMEM((ECH, D), jnp.float32),
        pltpu.VMEM((ECH, D), jnp.float32),
        pltpu.VMEM_SHARED((N, D), jnp.float32),
        pltpu.SemaphoreType.DMA,
        pltpu.SemaphoreType.DMA,
        pltpu.SemaphoreType.DMA,
        pltpu.SemaphoreType.DMA,
        pltpu.SemaphoreType.DMA,
        pltpu.SemaphoreType.DMA,
        pltpu.SemaphoreType.DMA,
        pltpu.SemaphoreType.DMA,
    ],
)
def _scatter_kernel(hs_hbm, src2_hbm, dst2_hbm, zeros_hbm, out_hbm,
                    src2_v, dst2_v, rows0, rows1, rows2, rows3, acc_sh,
                    gsem0, gsem1, gsem2, gsem3,
                    ssem0, ssem1, ssem2, ssem3):
    c = lax.axis_index("c")
    s = lax.axis_index("s")
    rows = (rows0, rows1, rows2, rows3)
    gsem = (gsem0, gsem1, gsem2, gsem3)
    ssem = (ssem0, ssem1, ssem2, ssem3)
    r0 = s * ROWS_PER_TILE
    b80 = (c * (E // NC) + s * EPT) // ECH
    for k in range(5):
        pltpu.async_copy(zeros_hbm, acc_sh.at[pl.ds(r0 + 125 * k, 125)],
                         gsem0)
    pltpu.async_copy(src2_hbm.at[pl.ds(b80, 45)], src2_v, gsem1)
    pltpu.async_copy(dst2_hbm.at[pl.ds(b80, 45)], dst2_v, gsem2)
    for k in range(5):
        pltpu.make_async_copy(zeros_hbm,
                              acc_sh.at[pl.ds(r0 + 125 * k, 125)],
                              gsem0).wait()
    pltpu.make_async_copy(src2_hbm.at[pl.ds(b80, 45)], src2_v,
                          gsem1).wait()
    pltpu.make_async_copy(dst2_hbm.at[pl.ds(b80, 45)], dst2_v,
                          gsem2).wait()
    plsc.subcore_barrier()

    def gather(ch, k):
        return pltpu.async_copy(hs_hbm.at[src2_v.at[ch]], rows[k], gsem[k])

    def gwait(ch, k):
        pltpu.make_async_copy(hs_hbm.at[src2_v.at[ch]], rows[k],
                              gsem[k]).wait()

    def scat(ch, k):
        return pltpu.async_copy(rows[k], acc_sh.at[dst2_v.at[ch]], ssem[k],
                                add=True)

    def swait(ch, k):
        pltpu.make_async_copy(rows[k], acc_sh.at[dst2_v.at[ch]],
                              ssem[k]).wait()

    def refill(goff, nch):
        pltpu.sync_copy(src2_hbm.at[pl.ds(b80 + goff, nch)],
                        src2_v.at[pl.ds(0, nch)])
        pltpu.sync_copy(dst2_hbm.at[pl.ds(b80 + goff, nch)],
                        dst2_v.at[pl.ds(0, nch)])

    def phase(o, P):
        for k in range(4):
            gather(o + k, k)

        def body(g, _):
            c0 = o + 4 * g
            for k in range(4):
                gwait(c0 + k, k)
                scat(c0 + k, k)

            @pl.when(g < P // 4 - 1)
            def _():
                for k in range(4):
                    swait(c0 + k, k)
                    gather(c0 + 4 + k, k)
            return 0

        lax.fori_loop(0, P // 4, body, 0)
        for k in range(4):
            swait(o + P - 4 + k, k)

    gather(0, 0).wait()
    scat(0, 0)
    swait(0, 0)
    phase(1, 44)
    refill(45, 40)
    phase(0, 40)
    refill(85, 40)
    phase(0, 40)
    plsc.subcore_barrier()
    pltpu.sync_copy(acc_sh.at[pl.ds(r0, ROWS_PER_TILE)],
                    out_hbm.at[pl.ds(c * N + r0, ROWS_PER_TILE)])


@functools.partial(
    pl.kernel,
    out_type=[
        jax.ShapeDtypeStruct((QPAD,), jnp.float32),
        jax.ShapeDtypeStruct((QPAD,), jnp.float32),
        jax.ShapeDtypeStruct((QPAD,), jnp.float32),
    ],
    mesh=_MESH,
    compiler_params=_SC_PARAMS,
    scratch_types=[
        pltpu.VMEM((N,), jnp.int32),
        pltpu.VMEM((2, QCH), jnp.int32),
        pltpu.VMEM((2, QCH), jnp.int32),
        pltpu.VMEM((2, 4, QCH), jnp.int32),
        pltpu.VMEM((2, 4, QCH, OUT), jnp.float32),
        pltpu.VMEM((QPT,), jnp.float32),
        pltpu.VMEM((QPT,), jnp.float32),
        pltpu.VMEM((QPT,), jnp.float32),
        pltpu.SemaphoreType.DMA,
        pltpu.SemaphoreType.DMA,
        pltpu.SemaphoreType.DMA,
        pltpu.SemaphoreType.DMA,
    ],
)
def _query_kernel(ef_hbm, y_hbm, qr_hbm, qc_hbm,
                  pi_hbm, pj_hbm, pr_hbm,
                  y_v, qr_v, qc_v, idx_v, rows_v, pi_v, pj_v, pr_v,
                  qsem0, qsem1, qdsem0, qdsem1):
    c = lax.axis_index("c")
    s = lax.axis_index("s")
    qsem = (qsem0, qsem1)
    qdsem = (qdsem0, qdsem1)
    wid = s * NC + c
    qbase = wid * QPT
    pltpu.sync_copy(y_hbm, y_v)

    def qdma(ch, k):
        qoff = qbase + ch * QCH
        pltpu.async_copy(qr_hbm.at[pl.ds(qoff, QCH)], qr_v.at[k], qdsem[k])
        pltpu.async_copy(qc_hbm.at[pl.ds(qoff, QCH)], qc_v.at[k], qdsem[k])

    def qwait(ch, k):
        qoff = qbase + ch * QCH
        pltpu.make_async_copy(qr_hbm.at[pl.ds(qoff, QCH)], qr_v.at[k],
                              qdsem[k]).wait()
        pltpu.make_async_copy(qc_hbm.at[pl.ds(qoff, QCH)], qc_v.at[k],
                              qdsem[k]).wait()

    def build_fire(k):
        for j in range(QCH // 16):
            qr16 = qr_v[k, pl.ds(16 * j, 16)]
            qc16 = qc_v[k, pl.ds(16 * j, 16)]
            yr = plsc.load_gather(y_v, [qr16])
            yc = plsc.load_gather(y_v, [qc16])
            idx_v[k, 0, pl.ds(16 * j, 16)] = qr16 * HEADS + yr
            idx_v[k, 1, pl.ds(16 * j, 16)] = qc16 * HEADS + yr
            idx_v[k, 2, pl.ds(16 * j, 16)] = qr16 * HEADS + yc
            idx_v[k, 3, pl.ds(16 * j, 16)] = qc16 * HEADS + yc
        for i in range(4):
            pltpu.async_copy(ef_hbm.at[idx_v.at[k, i]], rows_v.at[k, i],
                             qsem[k])

    def compute(ch, k):
        for i in range(4):
            pltpu.make_async_copy(ef_hbm.at[idx_v.at[k, i]],
                                  rows_v.at[k, i], qsem[k]).wait()
        k16 = jnp.full((16,), k, jnp.int32)
        i16 = [jnp.full((16,), i, jnp.int32) for i in range(4)]

        def jbody(j, _):
            row16 = lax.iota(jnp.int32, 16) + 16 * j
            acc_i = jnp.zeros((16,), jnp.float32)
            acc_j = jnp.zeros((16,), jnp.float32)
            for f in range(OUT):
                f16 = jnp.full((16,), f, jnp.int32)
                va = plsc.load_gather(rows_v, [k16, i16[0], row16, f16])
                vb = plsc.load_gather(rows_v, [k16, i16[1], row16, f16])
                vc = plsc.load_gather(rows_v, [k16, i16[2], row16, f16])
                vd = plsc.load_gather(rows_v, [k16, i16[3], row16, f16])
                acc_i = acc_i + va * vb
                acc_j = acc_j + vc * vd
            o = ch * QCH + 16 * j
            pi_v[pl.ds(o, 16)] = acc_i
            pj_v[pl.ds(o, 16)] = acc_j
            pr_v[pl.ds(o, 16)] = (acc_i + acc_j) * 0.5
            return 0

        lax.fori_loop(0, QCH // 16, jbody, 0)

    qdma(0, 0)
    qwait(0, 0)
    build_fire(0)
    qdma(1, 1)
    qwait(1, 1)
    build_fire(1)

    def body(gp, _):
        ch0 = 2 * gp

        @pl.when(gp < QNCH // 2 - 1)
        def _():
            qdma(ch0 + 2, 0)
        compute(ch0, 0)

        @pl.when(gp < QNCH // 2 - 1)
        def _():
            qwait(ch0 + 2, 0)
            build_fire(0)
            qdma(ch0 + 3, 1)
        compute(ch0 + 1, 1)

        @pl.when(gp < QNCH // 2 - 1)
        def _():
            qwait(ch0 + 3, 1)
            build_fire(1)
        return 0

    lax.fori_loop(0, QNCH // 2, body, 0)
    pltpu.sync_copy(pi_v, pi_hbm.at[pl.ds(qbase, QPT)])
    pltpu.sync_copy(pj_v, pj_hbm.at[pl.ds(qbase, QPT)])
    pltpu.sync_copy(pr_v, pr_hbm.at[pl.ds(qbase, QPT)])


_BLK = 2000
_GRID = N // _BLK


def _mm_scale_body(dp0_ref, dp1_ref, x_ref, w_ref, hs_ref, dinv_ref):
    deg = jnp.sum(dp0_ref[...] + dp1_ref[...], axis=1) * (1.0 / 16.0) + 1.0
    dinv = lax.rsqrt(deg)
    h = jnp.dot(x_ref[...], w_ref[...], preferred_element_type=jnp.float32)
    hs_ref[...] = h * dinv[:, None]
    dinv_ref[...] = dinv[:, None]


def _mid_body(a0_ref, a1_ref, hs_ref, dinv_ref, b_ref, w_ref,
              hid_ref, hs2_ref):
    dinv = dinv_ref[...]
    pre = (a0_ref[...] + a1_ref[...] + hs_ref[...]) * dinv + b_ref[...]
    hid = jnp.maximum(pre, 0.0)
    hid_ref[...] = hid
    h2 = jnp.dot(hid, w_ref[...], preferred_element_type=jnp.float32)
    hs2_ref[...] = h2 * dinv


def _emb_body(a0_ref, a1_ref, hs_ref, dinv_ref, b_ref, emb_ref):
    emb_ref[...] = (a0_ref[...] + a1_ref[...] + hs_ref[...]) * dinv_ref[...] \
        + b_ref[...]


def _tc_mm_scale(degp, x, w):
    return pl.pallas_call(
        _mm_scale_body,
        grid=(_GRID,),
        in_specs=[
            pl.BlockSpec((_BLK, 16), lambda g: (g, 0)),
            pl.BlockSpec((_BLK, 16), lambda g: (g + _GRID, 0)),
            pl.BlockSpec((_BLK, D), lambda g: (g, 0)),
            pl.BlockSpec((D, D), lambda g: (0, 0)),
        ],
        out_specs=[
            pl.BlockSpec((_BLK, D), lambda g: (g, 0)),
            pl.BlockSpec((_BLK, 1), lambda g: (g, 0)),
        ],
        out_shape=[
            jax.ShapeDtypeStruct((N, D), jnp.float32),
            jax.ShapeDtypeStruct((N, 1), jnp.float32),
        ],
    )(degp, degp, x, w)


def _tc_mid(accp, hs1, dinv, b1, w2):
    return pl.pallas_call(
        _mid_body,
        grid=(_GRID,),
        in_specs=[
            pl.BlockSpec((_BLK, D), lambda g: (g, 0)),
            pl.BlockSpec((_BLK, D), lambda g: (g + _GRID, 0)),
            pl.BlockSpec((_BLK, D), lambda g: (g, 0)),
            pl.BlockSpec((_BLK, 1), lambda g: (g, 0)),
            pl.BlockSpec((1, D), lambda g: (0, 0)),
            pl.BlockSpec((D, D), lambda g: (0, 0)),
        ],
        out_specs=[
            pl.BlockSpec((_BLK, D), lambda g: (g, 0)),
            pl.BlockSpec((_BLK, D), lambda g: (g, 0)),
        ],
        out_shape=[
            jax.ShapeDtypeStruct((N, D), jnp.float32),
            jax.ShapeDtypeStruct((N, D), jnp.float32),
        ],
    )(accp, accp, hs1, dinv, b1, w2)


def _tc_emb(accp, hs2, dinv, b2):
    return pl.pallas_call(
        _emb_body,
        grid=(_GRID,),
        in_specs=[
            pl.BlockSpec((_BLK, D), lambda g: (g, 0)),
            pl.BlockSpec((_BLK, D), lambda g: (g + _GRID, 0)),
            pl.BlockSpec((_BLK, D), lambda g: (g, 0)),
            pl.BlockSpec((_BLK, 1), lambda g: (g, 0)),
            pl.BlockSpec((1, D), lambda g: (0, 0)),
        ],
        out_specs=pl.BlockSpec((_BLK, D), lambda g: (g, 0)),
        out_shape=jax.ShapeDtypeStruct((N, D), jnp.float32),
    )(accp, accp, hs2, dinv, b2)


@jax.jit
def _run(x, edge_index, y, q_edge_index, W1, b1, W2, b2):
    src2 = edge_index[0].reshape(E // ECH, ECH)
    dst2 = edge_index[1].reshape(E // ECH, ECH)
    zeros = jnp.zeros((125, D), jnp.float32)
    zeros16 = jnp.zeros((125, 16), jnp.float32)
    ones16 = jnp.ones((ECH, 16), jnp.float32)

    degp = _deg_kernel(dst2, zeros16, ones16)
    hs1, dinv = _tc_mm_scale(degp, x, W1)
    acc1 = _scatter_kernel(hs1, src2, dst2, zeros)
    hiddens, hs2 = _tc_mid(acc1, hs1, dinv, b1.reshape(1, D), W2)
    acc2 = _scatter_kernel(hs2, src2, dst2, zeros)
    emb2d = _tc_emb(acc2, hs2, dinv, b2.reshape(1, D))

    ef = emb2d.reshape(N * HEADS, OUT)
    qpad = jnp.zeros((2, QPAD), jnp.int32).at[:, :Q].set(q_edge_index)
    pi, pj, pr = _query_kernel(ef, y, qpad[0], qpad[1])
    emb = emb2d.reshape(N, HEADS, OUT)
    return hiddens, emb, pi[:Q], pj[:Q], pr[:Q]


def kernel(x, edge_index, y, q_edge_index, W1, b1, W2, b2):
    return _run(x, edge_index, y, q_edge_index, W1, b1, W2, b2)
